# baseline (speedup 1.0000x reference)
import jax
import jax.numpy as jnp
from jax.experimental import pallas as pl
from jax.experimental.pallas import tpu as pltpu

_N_POINT = 1024
_T = 0.01
_VIDX = ((0, 1, 0, 2, 0, 3), (1, 2, 1, 3, 2, 3))


def _pair_kernel(vidx_ref, x_ref,
                 wf1_ref, bf1_ref, wf2_ref, bf2_ref,
                 wc1_ref, bc1_ref, wc2_ref, bc2_ref,
                 out_ref):
    p = pl.program_id(0)
    bf1r = bf1_ref[...].reshape(1, 64)
    bf2r = bf2_ref[...].reshape(1, 3)
    bc1r = bc1_ref[...].reshape(1, 64)
    bc2r = bc2_ref[...].reshape(1, 1)
    wf1t = wf1_ref[...].T
    wf2t = wf2_ref[...].T
    wc1t = wc1_ref[...].T
    wc2t = wc2_ref[...].T
    bf1c = bf1r.T
    bf2c = bf2r.T
    bc1c = bc1r.T
    for q in range(3):
        pc_i = x_ref[0, vidx_ref[p, 2 * q]]
        pc_j = x_ref[0, vidx_ref[p, 2 * q + 1]]
        _one_pair(pc_i, pc_j,
                  wf1_ref[...], bf1r, wf2_ref[...], bf2r,
                  wc1_ref[...], bc1r, wc2_ref[...], bc2r,
                  wf1t, bf1c, wf2t, bf2c,
                  wc1t, bc1c, wc2t,
                  out_ref, q)


def _one_pair(pc_i, pc_j,
              wf1, bf1r, wf2, bf2r, wc1, bc1r, wc2, bc2r,
              wf1t, bf1c, wf2t, bf2c, wc1t, bc1c, wc2t,
              out_ref, q):
    f32 = jnp.float32
    pc_jt = pc_j.T

    h_i = jnp.tanh(jnp.dot(pc_i, wf1, preferred_element_type=f32)
                   + bf1r)
    a_i = pc_i + jnp.dot(h_i, wf2, preferred_element_type=f32) \
        + bf2r
    hc_i = jnp.tanh(jnp.dot(a_i, wc1, preferred_element_type=f32)
                    + bc1r)
    w_i = jax.nn.sigmoid(
        jnp.dot(hc_i, wc2, preferred_element_type=f32)
        + bc2r)

    h_jt = jnp.tanh(jnp.dot(wf1t, pc_jt, preferred_element_type=f32)
                    + bf1c)
    b_jt = pc_jt + jnp.dot(wf2t, h_jt, preferred_element_type=f32) \
        + bf2c
    hc_jt = jnp.tanh(jnp.dot(wc1t, b_jt, preferred_element_type=f32)
                     + bc1c)
    w_j = jax.nn.sigmoid(
        jnp.dot(wc2t, hc_jt, preferred_element_type=f32)
        + bc2r)

    c = 1.4426950408889634 / _T
    aic = a_i * c
    pic = pc_i * c
    pjtc = pc_jt * c
    bjtc = b_jt * c
    dx12 = aic[:, 0:1] - pjtc[0:1, :]
    dy12 = aic[:, 1:2] - pjtc[1:2, :]
    dz12 = aic[:, 2:3] - pjtc[2:3, :]
    s12 = (dx12 * dx12 + dy12 * dy12) + (dz12 * dz12 + 1e-24)
    d12 = s12 * jax.lax.rsqrt(s12)
    dx21 = pic[:, 0:1] - bjtc[0:1, :]
    dy21 = pic[:, 1:2] - bjtc[1:2, :]
    dz21 = pic[:, 2:3] - bjtc[2:3, :]
    s21 = (dx21 * dx21 + dy21 * dy21) + (dz21 * dz21 + 1e-24)
    d21 = s21 * jax.lax.rsqrt(s21)

    u = w_j / (w_i + w_j)
    blend = d12 + (d21 - d12) * u
    mb = jnp.min(blend, axis=1, keepdims=True)
    e = jnp.exp2(mb - blend)
    out_ref[q, 0] = e * (1.0 / jnp.sum(e, axis=1, keepdims=True))


def kernel(xyz, Wf1, bf1, Wf2, bf2, Wc1, bc1, Wc2, bc2):
    vidx = jnp.asarray(_VIDX, dtype=jnp.int32)

    full = lambda shape: pl.BlockSpec(shape, lambda p, v: (0,) * len(shape))
    grid_spec = pltpu.PrefetchScalarGridSpec(
        num_scalar_prefetch=1,
        grid=(2,),
        in_specs=[
            full((1, 4, _N_POINT, 3)),
            full((3, 64)), full((64,)),
            full((64, 3)), full((3,)),
            full((3, 64)), full((64,)),
            full((64, 1)), full((1,)),
        ],
        out_specs=pl.BlockSpec((3, 1, _N_POINT, _N_POINT),
                               lambda p, v: (p, 0, 0, 0)),
    )
    return pl.pallas_call(
        _pair_kernel,
        grid_spec=grid_spec,
        out_shape=jax.ShapeDtypeStruct((6, 1, _N_POINT, _N_POINT),
                                       jnp.float32),
        compiler_params=pltpu.CompilerParams(
            dimension_semantics=("parallel",)),
    )(
        vidx, xyz,
        Wf1, bf1, Wf2, bf2, Wc1, bc1, Wc2, bc2,
    )
